# baseline (speedup 1.0000x reference)
import jax
import jax.numpy as jnp
from jax.experimental import pallas as pl
from jax.experimental.pallas import tpu as pltpu

_BB = 64


def _fused_kernel(x_ref, w1t_ref, b1t_ref, w2t_ref, b2t_ref, w3t_ref,
                  b3t_ref, wf1_ref, bf1_ref, wf2_ref, bf2_ref, o_ref):
    Bb = x_ref.shape[0]
    x = x_ref[...]

    x1 = jnp.concatenate([x[:, 0:26, :], x[:, 1:27, :], x[:, 2:28, :]],
                         axis=-1)
    y1 = jnp.dot(x1.reshape(Bb * 26, 84), w1t_ref[...],
                 preferred_element_type=jnp.float32) + b1t_ref[...]
    y1 = jnp.maximum(y1, 0.0).astype(jnp.bfloat16).reshape(Bb, 26, 832)

    y1 = y1.reshape(Bb, 13, 2, 832).max(axis=2)
    ym = jnp.maximum(y1[..., :800], y1[..., 32:])
    p1 = jnp.concatenate([ym[..., 64 * r:64 * r + 32] for r in range(13)],
                         axis=-1)

    x2 = jnp.concatenate([p1[:, 0:11, :], p1[:, 1:12, :], p1[:, 2:13, :]],
                         axis=-1)
    y2 = jnp.dot(x2.reshape(Bb * 11, 1248), w2t_ref[...],
                 preferred_element_type=jnp.float32) + b2t_ref[...]
    y2 = jnp.maximum(y2, 0.0).astype(jnp.bfloat16).reshape(Bb, 11, 704)

    y2 = y2[:, 0:10, :].reshape(Bb, 5, 2, 704).max(axis=2)
    ym2 = jnp.maximum(y2[..., :640], y2[..., 64:])
    p2 = jnp.concatenate([ym2[..., 128 * r:128 * r + 64] for r in range(5)],
                         axis=-1)

    x3 = jnp.concatenate([p2[:, 0:3, :], p2[:, 1:4, :], p2[:, 2:5, :]],
                         axis=-1)
    y3 = jnp.dot(x3.reshape(Bb * 3, 960), w3t_ref[...],
                 preferred_element_type=jnp.float32) + b3t_ref[...]
    y3 = jnp.maximum(y3, 0.0).astype(jnp.bfloat16).reshape(Bb, 3, 384)

    yh = jnp.maximum(y3[:, 0, :], y3[:, 1, :])
    f = jnp.maximum(yh[:, :128], yh[:, 128:256])

    h = jnp.maximum(jnp.dot(f, wf1_ref[...],
                            preferred_element_type=jnp.float32)
                    + bf1_ref[...], 0.0).astype(jnp.bfloat16)
    logits = (jnp.dot(h, wf2_ref[...], preferred_element_type=jnp.float32)
              + bf2_ref[...])
    m = jnp.max(logits, axis=-1, keepdims=True)
    s = logits - m
    lse = jnp.log(jnp.sum(jnp.exp(s), axis=-1, keepdims=True))
    o_ref[...] = (s - lse).astype(o_ref.dtype)


def _toeplitz_w(w, win, wout, cin, cout):
    diff = jnp.arange(win)[:, None] - jnp.arange(wout)[None, :]
    mask = (diff >= 0) & (diff <= 2)
    idx = jnp.clip(diff, 0, 2)
    wt = w[:, idx]
    wt = jnp.where(mask[None, :, :, None, None], wt, 0.0)
    wt = wt.transpose(0, 1, 3, 2, 4)
    return wt.reshape(3 * win * cin, wout * cout)


def kernel(x_nchw, conv1_w, conv1_b, conv2_w, conv2_b, conv3_w, conv3_b,
           fc1_w, fc1_b, fc2_w, fc2_b):
    N, C, H, W = x_nchw.shape
    assert (C, H, W) == (1, 28, 28), (C, H, W)
    Bb = _BB
    n_pad = int(pl.cdiv(N, Bb)) * Bb

    x = x_nchw.reshape(N, 28, 28)
    if n_pad != N:
        x = jnp.pad(x, ((0, n_pad - N), (0, 0), (0, 0)))
    x = x.astype(jnp.bfloat16)

    w1t = _toeplitz_w(conv1_w.reshape(3, 3, 1, 32), 28, 26, 1, 32).astype(jnp.bfloat16)
    b1t = jnp.tile(conv1_b.reshape(1, 32), (1, 26))
    w2t = _toeplitz_w(conv2_w, 13, 11, 32, 64).astype(jnp.bfloat16)
    b2t = jnp.tile(conv2_b.reshape(1, 64), (1, 11))
    w3t = _toeplitz_w(conv3_w, 5, 3, 64, 128).astype(jnp.bfloat16)
    b3t = jnp.tile(conv3_b.reshape(1, 128), (1, 3))
    wf1 = fc1_w.reshape(128, 9, 512).sum(axis=1).astype(jnp.bfloat16)
    bf1 = fc1_b.reshape(1, 512)
    wf2 = fc2_w.astype(jnp.bfloat16)
    bf2 = fc2_b.reshape(1, 10)

    def const2d(shape):
        return pl.BlockSpec(shape, lambda g: (0, 0))

    out = pl.pallas_call(
        _fused_kernel,
        out_shape=jax.ShapeDtypeStruct((n_pad, 10), jnp.float32),
        grid=(n_pad // Bb,),
        in_specs=[
            pl.BlockSpec((Bb, 28, 28), lambda g: (g, 0, 0)),
            const2d((84, 832)),
            const2d((1, 832)),
            const2d((1248, 704)),
            const2d((1, 704)),
            const2d((960, 384)),
            const2d((1, 384)),
            const2d((128, 512)),
            const2d((1, 512)),
            const2d((512, 10)),
            const2d((1, 10)),
        ],
        out_specs=pl.BlockSpec((Bb, 10), lambda g: (g, 0)),
        compiler_params=pltpu.CompilerParams(
            dimension_semantics=("parallel",),
            vmem_limit_bytes=60 * 1024 * 1024,
        ),
    )(x, w1t, b1t, w2t, b2t, w3t, b3t, wf1, bf1, wf2, bf2)
    return out[:N]


# speedup vs baseline: 1.1928x; 1.1928x over previous
import jax
import jax.numpy as jnp
from jax.experimental import pallas as pl
from jax.experimental.pallas import tpu as pltpu

_BB = 64


def _fused_kernel(x_ref, w1t_ref, b1t_ref, w2t_ref, b2t_ref, w3t_ref,
                  b3t_ref, wf1_ref, bf1_ref, wf2_ref, bf2_ref, o_ref):
    Bb = x_ref.shape[0]
    x = x_ref[...]

    x1 = jnp.concatenate([x[:, 0:26, :], x[:, 1:27, :], x[:, 2:28, :]],
                         axis=-1)
    y1 = jnp.dot(x1.reshape(Bb * 26, 84), w1t_ref[...],
                 preferred_element_type=jnp.float32) + b1t_ref[...]
    y1 = jnp.maximum(y1, 0.0).reshape(Bb, 26, 832)

    y1 = y1.reshape(Bb, 13, 2, 832).max(axis=2)
    ym = jnp.maximum(y1[..., :800], y1[..., 32:])
    p1 = jnp.concatenate([ym[..., 64 * r:64 * r + 32] for r in range(13)],
                         axis=-1)

    x2 = jnp.concatenate([p1[:, 0:11, :], p1[:, 1:12, :], p1[:, 2:13, :]],
                         axis=-1)
    y2 = jnp.dot(x2.reshape(Bb * 11, 1248), w2t_ref[...],
                 preferred_element_type=jnp.float32) + b2t_ref[...]
    y2 = jnp.maximum(y2, 0.0).reshape(Bb, 11, 704)

    y2 = y2[:, 0:10, :].reshape(Bb, 5, 2, 704).max(axis=2)
    ym2 = jnp.maximum(y2[..., :640], y2[..., 64:])
    p2 = jnp.concatenate([ym2[..., 128 * r:128 * r + 64] for r in range(5)],
                         axis=-1)

    x3 = jnp.concatenate([p2[:, 0:3, :], p2[:, 1:4, :], p2[:, 2:5, :]],
                         axis=-1)
    y3 = jnp.dot(x3.reshape(Bb * 3, 960), w3t_ref[...],
                 preferred_element_type=jnp.float32) + b3t_ref[...]
    y3 = jnp.maximum(y3, 0.0).reshape(Bb, 3, 384)

    yh = jnp.maximum(y3[:, 0, :], y3[:, 1, :])
    f = jnp.maximum(yh[:, :128], yh[:, 128:256])

    h = jnp.maximum(jnp.dot(f, wf1_ref[...],
                            preferred_element_type=jnp.float32)
                    + bf1_ref[...], 0.0)
    logits = (jnp.dot(h, wf2_ref[...], preferred_element_type=jnp.float32)
              + bf2_ref[...])
    m = jnp.max(logits, axis=-1, keepdims=True)
    s = logits - m
    lse = jnp.log(jnp.sum(jnp.exp(s), axis=-1, keepdims=True))
    o_ref[...] = (s - lse).astype(o_ref.dtype)


def _toeplitz_w(w, win, wout, cin, cout):
    diff = jnp.arange(win)[:, None] - jnp.arange(wout)[None, :]
    mask = (diff >= 0) & (diff <= 2)
    idx = jnp.clip(diff, 0, 2)
    wt = w[:, idx]
    wt = jnp.where(mask[None, :, :, None, None], wt, 0.0)
    wt = wt.transpose(0, 1, 3, 2, 4)
    return wt.reshape(3 * win * cin, wout * cout)


def kernel(x_nchw, conv1_w, conv1_b, conv2_w, conv2_b, conv3_w, conv3_b,
           fc1_w, fc1_b, fc2_w, fc2_b):
    N, C, H, W = x_nchw.shape
    assert (C, H, W) == (1, 28, 28), (C, H, W)
    Bb = _BB
    n_pad = int(pl.cdiv(N, Bb)) * Bb

    x = x_nchw.reshape(N, 28, 28)
    if n_pad != N:
        x = jnp.pad(x, ((0, n_pad - N), (0, 0), (0, 0)))

    w1t = _toeplitz_w(conv1_w.reshape(3, 3, 1, 32), 28, 26, 1, 32)
    b1t = jnp.tile(conv1_b.reshape(1, 32), (1, 26))
    w2t = _toeplitz_w(conv2_w, 13, 11, 32, 64)
    b2t = jnp.tile(conv2_b.reshape(1, 64), (1, 11))
    w3t = _toeplitz_w(conv3_w, 5, 3, 64, 128)
    b3t = jnp.tile(conv3_b.reshape(1, 128), (1, 3))
    wf1 = fc1_w.reshape(128, 9, 512).sum(axis=1)
    bf1 = fc1_b.reshape(1, 512)
    wf2 = fc2_w
    bf2 = fc2_b.reshape(1, 10)

    def const2d(shape):
        return pl.BlockSpec(shape, lambda g: (0, 0))

    out = pl.pallas_call(
        _fused_kernel,
        out_shape=jax.ShapeDtypeStruct((n_pad, 10), jnp.float32),
        grid=(n_pad // Bb,),
        in_specs=[
            pl.BlockSpec((Bb, 28, 28), lambda g: (g, 0, 0)),
            const2d((84, 832)),
            const2d((1, 832)),
            const2d((1248, 704)),
            const2d((1, 704)),
            const2d((960, 384)),
            const2d((1, 384)),
            const2d((128, 512)),
            const2d((1, 512)),
            const2d((512, 10)),
            const2d((1, 10)),
        ],
        out_specs=pl.BlockSpec((Bb, 10), lambda g: (g, 0)),
        compiler_params=pltpu.CompilerParams(
            dimension_semantics=("parallel",),
            vmem_limit_bytes=60 * 1024 * 1024,
        ),
    )(x, w1t, b1t, w2t, b2t, w3t, b3t, wf1, bf1, wf2, bf2)
    return out[:N]


# speedup vs baseline: 2.6976x; 2.2615x over previous
import jax
import jax.numpy as jnp
from jax.experimental import pallas as pl
from jax.experimental.pallas import tpu as pltpu

_BB = 64


def _fused_kernel(x0_ref, x1_ref, x2_ref, x3_ref, w1c_ref, b1t_ref,
                  w2c_ref, b2t_ref, w3c_ref, b3_ref, wf1_ref, bf1_ref,
                  wf2_ref, bf2_ref, o_ref):
    Bb = x0_ref.shape[0]
    xq = [x0_ref[...], x1_ref[...], x2_ref[...], x3_ref[...]]

    def dotf(a, w):
        return jnp.dot(a, w, preferred_element_type=jnp.float32)

    w1c = w1c_ref[...]
    b1t = b1t_ref[...]
    xa = jnp.concatenate([xq[0][:, 0:5], xq[1][:, 0:5], xq[2][:, 0:5]], -1)
    xb = jnp.concatenate([xq[1][:, 0:5], xq[2][:, 0:5], xq[3][:, 0:5]], -1)
    xc = jnp.concatenate([xq[2][:, 0:5], xq[3][:, 0:5], xq[0][:, 1:6]], -1)
    xd = jnp.concatenate([xq[3][:, 0:5], xq[0][:, 1:6], xq[1][:, 1:6]], -1)
    ya = dotf(xa.reshape(Bb * 5, 84), w1c)
    yb = dotf(xb.reshape(Bb * 5, 84), w1c)
    yc = dotf(xc.reshape(Bb * 5, 84), w1c)
    yd = dotf(xd.reshape(Bb * 5, 84), w1c)
    pe = jnp.maximum(jnp.maximum(ya[:, 0:320], ya[:, 384:704]),
                     jnp.maximum(yb[:, 0:320], yb[:, 384:704]))
    po = jnp.maximum(jnp.maximum(yc[:, 0:320], yc[:, 384:704]),
                     jnp.maximum(yd[:, 0:320], yd[:, 384:704]))
    p1e = jnp.maximum(pe + b1t, 0.0).reshape(Bb, 5, 320)
    p1o = jnp.maximum(po + b1t, 0.0).reshape(Bb, 5, 320)

    w2c = w2c_ref[...]
    b2t = b2t_ref[...]
    x2a = jnp.concatenate([p1e[:, 0:4], p1o[:, 0:4], p1e[:, 1:5]], -1)
    x2b = jnp.concatenate([p1o[:, 0:4], p1e[:, 1:5], p1o[:, 1:5]], -1)
    y2a = dotf(x2a.reshape(Bb * 4, 960), w2c)
    y2b = dotf(x2b.reshape(Bb * 4, 960), w2c)
    q2 = jnp.maximum(jnp.maximum(y2a[:, 0:256], y2a[:, 256:512]),
                     jnp.maximum(y2b[:, 0:256], y2b[:, 256:512]))
    p2 = jnp.maximum(q2 + b2t, 0.0).reshape(Bb, 4, 256)

    w3c = w3c_ref[...]
    x3 = jnp.concatenate([p2[:, 0:2], p2[:, 1:3], p2[:, 2:4]], -1)
    y3 = dotf(x3.reshape(Bb * 2, 768), w3c).reshape(Bb, 2, 256)
    v = jnp.maximum(y3[:, 0, :], y3[:, 1, :])
    f = jnp.maximum(jnp.maximum(v[:, 0:128], v[:, 128:256])
                    + b3_ref[...], 0.0)

    h = jnp.maximum(dotf(f, wf1_ref[...]) + bf1_ref[...], 0.0)
    logits = dotf(h, wf2_ref[...]) + bf2_ref[...]
    m = jnp.max(logits, axis=-1, keepdims=True)
    s = logits - m
    lse = jnp.log(jnp.sum(jnp.exp(s), axis=-1, keepdims=True))
    o_ref[...] = (s - lse).astype(o_ref.dtype)


def _toeplitz_w(w, win, wout, cin, cout):
    diff = jnp.arange(win)[:, None] - jnp.arange(wout)[None, :]
    mask = (diff >= 0) & (diff <= 2)
    idx = jnp.clip(diff, 0, 2)
    wt = w[:, idx]
    wt = jnp.where(mask[None, :, :, None, None], wt, 0.0)
    wt = wt.transpose(0, 1, 3, 2, 4)
    return wt.reshape(3 * win * cin, wout * cout)


def _parity_pack(wt, wout, cout, off_odd, n_total):
    k = wt.shape[0]
    w3 = wt.reshape(k, wout, cout)
    we = w3[:, 0::2].reshape(k, -1)
    wo = w3[:, 1::2].reshape(k, -1)
    out = jnp.zeros((k, n_total), wt.dtype)
    out = out.at[:, 0:we.shape[1]].set(we)
    out = out.at[:, off_odd:off_odd + wo.shape[1]].set(wo)
    return out


def kernel(x_nchw, conv1_w, conv1_b, conv2_w, conv2_b, conv3_w, conv3_b,
           fc1_w, fc1_b, fc2_w, fc2_b):
    N, C, H, W = x_nchw.shape
    assert (C, H, W) == (1, 28, 28), (C, H, W)
    Bb = _BB
    n_pad = int(pl.cdiv(N, Bb)) * Bb

    x = x_nchw.reshape(N, 28, 28)
    if n_pad != N:
        x = jnp.pad(x, ((0, n_pad - N), (0, 0), (0, 0)))
    xqs = [x[:, r::4, :] for r in range(4)]

    w1t = _toeplitz_w(conv1_w.reshape(3, 3, 1, 32), 28, 26, 1, 32)
    w1t = w1t.reshape(84, 26, 32)[:, 0:20].reshape(84, 640)
    w1c = _parity_pack(w1t, 20, 32, 384, 704)
    b1t = jnp.tile(conv1_b.reshape(1, 32), (1, 10))
    w2t = _toeplitz_w(conv2_w, 10, 8, 32, 64)
    w2c = _parity_pack(w2t, 8, 64, 256, 512)
    b2t = jnp.tile(conv2_b.reshape(1, 64), (1, 4))
    w3c = _toeplitz_w(conv3_w, 4, 3, 64, 128)[:, 0:256]
    b3 = conv3_b.reshape(1, 128)
    wf1 = fc1_w.reshape(128, 9, 512).sum(axis=1)
    bf1 = fc1_b.reshape(1, 512)
    wf2 = fc2_w
    bf2 = fc2_b.reshape(1, 10)

    def const2d(shape):
        return pl.BlockSpec(shape, lambda g: (0, 0))

    out = pl.pallas_call(
        _fused_kernel,
        out_shape=jax.ShapeDtypeStruct((n_pad, 10), jnp.float32),
        grid=(n_pad // Bb,),
        in_specs=[
            pl.BlockSpec((Bb, 7, 28), lambda g: (g, 0, 0)),
            pl.BlockSpec((Bb, 7, 28), lambda g: (g, 0, 0)),
            pl.BlockSpec((Bb, 7, 28), lambda g: (g, 0, 0)),
            pl.BlockSpec((Bb, 7, 28), lambda g: (g, 0, 0)),
            const2d((84, 704)),
            const2d((1, 320)),
            const2d((960, 512)),
            const2d((1, 256)),
            const2d((768, 256)),
            const2d((1, 128)),
            const2d((128, 512)),
            const2d((1, 512)),
            const2d((512, 10)),
            const2d((1, 10)),
        ],
        out_specs=pl.BlockSpec((Bb, 10), lambda g: (g, 0)),
        compiler_params=pltpu.CompilerParams(
            dimension_semantics=("parallel",),
            vmem_limit_bytes=60 * 1024 * 1024,
        ),
    )(*xqs, w1c, b1t, w2c, b2t, w3c, b3, wf1, bf1, wf2, bf2)
    return out[:N]


# speedup vs baseline: 2.8288x; 1.0486x over previous
import jax
import jax.numpy as jnp
from jax.experimental import pallas as pl
from jax.experimental.pallas import tpu as pltpu

_BB = 128


def _fused_kernel(x0_ref, x1_ref, x2_ref, x3_ref, w1c_ref, b1t_ref,
                  w2c_ref, b2t_ref, w3c_ref, b3_ref, wf1_ref, bf1_ref,
                  wf2_ref, bf2_ref, o_ref):
    Bb = x0_ref.shape[0]
    xq = [x0_ref[...], x1_ref[...], x2_ref[...], x3_ref[...]]

    def dotf(a, w):
        return jnp.dot(a, w, preferred_element_type=jnp.float32)

    w1c = w1c_ref[...]
    b1t = b1t_ref[...]
    xa = jnp.concatenate([xq[0][:, 0:5], xq[1][:, 0:5], xq[2][:, 0:5]], -1)
    xb = jnp.concatenate([xq[1][:, 0:5], xq[2][:, 0:5], xq[3][:, 0:5]], -1)
    xc = jnp.concatenate([xq[2][:, 0:5], xq[3][:, 0:5], xq[0][:, 1:6]], -1)
    xd = jnp.concatenate([xq[3][:, 0:5], xq[0][:, 1:6], xq[1][:, 1:6]], -1)
    ya = dotf(xa.reshape(Bb * 5, 84), w1c)
    yb = dotf(xb.reshape(Bb * 5, 84), w1c)
    yc = dotf(xc.reshape(Bb * 5, 84), w1c)
    yd = dotf(xd.reshape(Bb * 5, 84), w1c)
    pe = jnp.maximum(jnp.maximum(ya[:, 0:320], ya[:, 384:704]),
                     jnp.maximum(yb[:, 0:320], yb[:, 384:704]))
    po = jnp.maximum(jnp.maximum(yc[:, 0:320], yc[:, 384:704]),
                     jnp.maximum(yd[:, 0:320], yd[:, 384:704]))
    p1e = jnp.maximum(pe + b1t, 0.0).reshape(Bb, 5, 320)
    p1o = jnp.maximum(po + b1t, 0.0).reshape(Bb, 5, 320)

    w2c = w2c_ref[...]
    b2t = b2t_ref[...]
    x2a = jnp.concatenate([p1e[:, 0:4], p1o[:, 0:4], p1e[:, 1:5]], -1)
    x2b = jnp.concatenate([p1o[:, 0:4], p1e[:, 1:5], p1o[:, 1:5]], -1)
    y2a = dotf(x2a.reshape(Bb * 4, 960), w2c)
    y2b = dotf(x2b.reshape(Bb * 4, 960), w2c)
    q2 = jnp.maximum(jnp.maximum(y2a[:, 0:256], y2a[:, 256:512]),
                     jnp.maximum(y2b[:, 0:256], y2b[:, 256:512]))
    p2 = jnp.maximum(q2 + b2t, 0.0).reshape(Bb, 4, 256)

    w3c = w3c_ref[...]
    x3 = jnp.concatenate([p2[:, 0:2], p2[:, 1:3], p2[:, 2:4]], -1)
    y3 = dotf(x3.reshape(Bb * 2, 768), w3c).reshape(Bb, 2, 256)
    v = jnp.maximum(y3[:, 0, :], y3[:, 1, :])
    f = jnp.maximum(jnp.maximum(v[:, 0:128], v[:, 128:256])
                    + b3_ref[...], 0.0)

    h = jnp.maximum(dotf(f, wf1_ref[...]) + bf1_ref[...], 0.0)
    logits = dotf(h, wf2_ref[...]) + bf2_ref[...]
    m = jnp.max(logits, axis=-1, keepdims=True)
    s = logits - m
    lse = jnp.log(jnp.sum(jnp.exp(s), axis=-1, keepdims=True))
    o_ref[...] = (s - lse).astype(o_ref.dtype)


def _toeplitz_w(w, win, wout, cin, cout):
    diff = jnp.arange(win)[:, None] - jnp.arange(wout)[None, :]
    mask = (diff >= 0) & (diff <= 2)
    idx = jnp.clip(diff, 0, 2)
    wt = w[:, idx]
    wt = jnp.where(mask[None, :, :, None, None], wt, 0.0)
    wt = wt.transpose(0, 1, 3, 2, 4)
    return wt.reshape(3 * win * cin, wout * cout)


def _parity_pack(wt, wout, cout, off_odd, n_total):
    k = wt.shape[0]
    w3 = wt.reshape(k, wout, cout)
    we = w3[:, 0::2].reshape(k, -1)
    wo = w3[:, 1::2].reshape(k, -1)
    out = jnp.zeros((k, n_total), wt.dtype)
    out = out.at[:, 0:we.shape[1]].set(we)
    out = out.at[:, off_odd:off_odd + wo.shape[1]].set(wo)
    return out


def kernel(x_nchw, conv1_w, conv1_b, conv2_w, conv2_b, conv3_w, conv3_b,
           fc1_w, fc1_b, fc2_w, fc2_b):
    N, C, H, W = x_nchw.shape
    assert (C, H, W) == (1, 28, 28), (C, H, W)
    Bb = _BB
    n_pad = int(pl.cdiv(N, Bb)) * Bb

    x = x_nchw.reshape(N, 28, 28)
    if n_pad != N:
        x = jnp.pad(x, ((0, n_pad - N), (0, 0), (0, 0)))
    xqs = [x[:, r::4, :] for r in range(4)]

    w1t = _toeplitz_w(conv1_w.reshape(3, 3, 1, 32), 28, 26, 1, 32)
    w1t = w1t.reshape(84, 26, 32)[:, 0:20].reshape(84, 640)
    w1c = _parity_pack(w1t, 20, 32, 384, 704)
    b1t = jnp.tile(conv1_b.reshape(1, 32), (1, 10))
    w2t = _toeplitz_w(conv2_w, 10, 8, 32, 64)
    w2c = _parity_pack(w2t, 8, 64, 256, 512)
    b2t = jnp.tile(conv2_b.reshape(1, 64), (1, 4))
    w3c = _toeplitz_w(conv3_w, 4, 3, 64, 128)[:, 0:256]
    b3 = conv3_b.reshape(1, 128)
    wf1 = fc1_w.reshape(128, 9, 512).sum(axis=1)
    bf1 = fc1_b.reshape(1, 512)
    wf2 = fc2_w
    bf2 = fc2_b.reshape(1, 10)

    def const2d(shape):
        return pl.BlockSpec(shape, lambda g: (0, 0))

    out = pl.pallas_call(
        _fused_kernel,
        out_shape=jax.ShapeDtypeStruct((n_pad, 10), jnp.float32),
        grid=(n_pad // Bb,),
        in_specs=[
            pl.BlockSpec((Bb, 7, 28), lambda g: (g, 0, 0)),
            pl.BlockSpec((Bb, 7, 28), lambda g: (g, 0, 0)),
            pl.BlockSpec((Bb, 7, 28), lambda g: (g, 0, 0)),
            pl.BlockSpec((Bb, 7, 28), lambda g: (g, 0, 0)),
            const2d((84, 704)),
            const2d((1, 320)),
            const2d((960, 512)),
            const2d((1, 256)),
            const2d((768, 256)),
            const2d((1, 128)),
            const2d((128, 512)),
            const2d((1, 512)),
            const2d((512, 10)),
            const2d((1, 10)),
        ],
        out_specs=pl.BlockSpec((Bb, 10), lambda g: (g, 0)),
        compiler_params=pltpu.CompilerParams(
            dimension_semantics=("parallel",),
            vmem_limit_bytes=60 * 1024 * 1024,
        ),
    )(*xqs, w1c, b1t, w2c, b2t, w3c, b3, wf1, bf1, wf2, bf2)
    return out[:N]


# speedup vs baseline: 5.9282x; 2.0957x over previous
import jax
import jax.numpy as jnp
from jax.experimental import pallas as pl
from jax.experimental.pallas import tpu as pltpu

_BB = 128


def _fused_kernel(x0_ref, x1_ref, x2_ref, x3_ref, w1c_ref, b1t_ref,
                  w2c_ref, b2t_ref, w3c_ref, b3_ref, wf1_ref, bf1_ref,
                  wf2_ref, bf2_ref, o_ref):
    Bb = x0_ref.shape[1]
    xq = [x0_ref[...], x1_ref[...], x2_ref[...], x3_ref[...]]

    def dotf(a, w):
        return jnp.dot(a, w, preferred_element_type=jnp.float32)

    w1c = w1c_ref[...]
    b1t = b1t_ref[...]
    xa = jnp.concatenate([xq[0][0:5], xq[1][0:5], xq[2][0:5]], -1)
    xb = jnp.concatenate([xq[1][0:5], xq[2][0:5], xq[3][0:5]], -1)
    xc = jnp.concatenate([xq[2][0:5], xq[3][0:5], xq[0][1:6]], -1)
    xd = jnp.concatenate([xq[3][0:5], xq[0][1:6], xq[1][1:6]], -1)
    ya = dotf(xa.reshape(5 * Bb, 384), w1c)
    yb = dotf(xb.reshape(5 * Bb, 384), w1c)
    yc = dotf(xc.reshape(5 * Bb, 384), w1c)
    yd = dotf(xd.reshape(5 * Bb, 384), w1c)
    pe = jnp.maximum(jnp.maximum(ya[:, 0:384], ya[:, 384:768]),
                     jnp.maximum(yb[:, 0:384], yb[:, 384:768]))
    po = jnp.maximum(jnp.maximum(yc[:, 0:384], yc[:, 384:768]),
                     jnp.maximum(yd[:, 0:384], yd[:, 384:768]))
    p1e = jnp.maximum(pe + b1t, 0.0)
    p1o = jnp.maximum(po + b1t, 0.0)

    w2c = w2c_ref[...]
    b2t = b2t_ref[...]
    x2a = jnp.concatenate([p1e[0:4 * Bb], p1o[0:4 * Bb], p1e[Bb:5 * Bb]], -1)
    x2b = jnp.concatenate([p1o[0:4 * Bb], p1e[Bb:5 * Bb], p1o[Bb:5 * Bb]], -1)
    y2a = dotf(x2a, w2c)
    y2b = dotf(x2b, w2c)
    q2 = jnp.maximum(jnp.maximum(y2a[:, 0:256], y2a[:, 256:512]),
                     jnp.maximum(y2b[:, 0:256], y2b[:, 256:512]))
    p2 = jnp.maximum(q2 + b2t, 0.0)

    w3c = w3c_ref[...]
    x3 = jnp.concatenate([p2[0:2 * Bb], p2[Bb:3 * Bb], p2[2 * Bb:4 * Bb]], -1)
    y3 = dotf(x3, w3c)
    v = jnp.maximum(y3[0:Bb], y3[Bb:2 * Bb])
    f = jnp.maximum(jnp.maximum(v[:, 0:128], v[:, 128:256])
                    + b3_ref[...], 0.0)

    h = jnp.maximum(dotf(f, wf1_ref[...]) + bf1_ref[...], 0.0)
    logits = dotf(h, wf2_ref[...]) + bf2_ref[...]
    m = jnp.max(logits, axis=-1, keepdims=True)
    s = logits - m
    lse = jnp.log(jnp.sum(jnp.exp(s), axis=-1, keepdims=True))
    o_ref[...] = (s - lse).astype(o_ref.dtype)


def _toeplitz_w(w, win, wout, cin, cout):
    diff = jnp.arange(win)[:, None] - jnp.arange(wout)[None, :]
    mask = (diff >= 0) & (diff <= 2)
    idx = jnp.clip(diff, 0, 2)
    wt = w[:, idx]
    wt = jnp.where(mask[None, :, :, None, None], wt, 0.0)
    wt = wt.transpose(0, 1, 3, 2, 4)
    return wt.reshape(3 * win * cin, wout * cout)


def _parity_pack(wt, wout, cout, off_odd, n_total):
    k = wt.shape[0]
    w3 = wt.reshape(k, wout, cout)
    we = w3[:, 0::2].reshape(k, -1)
    wo = w3[:, 1::2].reshape(k, -1)
    out = jnp.zeros((k, n_total), wt.dtype)
    out = out.at[:, 0:we.shape[1]].set(we)
    out = out.at[:, off_odd:off_odd + wo.shape[1]].set(wo)
    return out


def kernel(x_nchw, conv1_w, conv1_b, conv2_w, conv2_b, conv3_w, conv3_b,
           fc1_w, fc1_b, fc2_w, fc2_b):
    N, C, H, W = x_nchw.shape
    assert (C, H, W) == (1, 28, 28), (C, H, W)
    Bb = _BB
    n_pad = int(pl.cdiv(N, Bb)) * Bb

    x = x_nchw.reshape(N, 28, 28)
    if n_pad != N:
        x = jnp.pad(x, ((0, n_pad - N), (0, 0), (0, 0)))
    xqs = [jnp.pad(x[:, r::4, :], ((0, 0), (0, 0), (0, 100))).transpose(1, 0, 2)
           for r in range(4)]

    w1t = _toeplitz_w(conv1_w.reshape(3, 3, 1, 32), 28, 26, 1, 32)
    w1t = w1t.reshape(84, 26, 32)[:, 0:20].reshape(84, 640)
    w1p = _parity_pack(w1t, 20, 32, 384, 768)
    w1c = jnp.zeros((3, 128, 768), w1p.dtype).at[:, 0:28].set(
        w1p.reshape(3, 28, 768)).reshape(384, 768)
    b1t = jnp.zeros((1, 384), jnp.float32).at[:, 0:320].set(
        jnp.tile(conv1_b.reshape(1, 32), (1, 10)))
    w2c = _parity_pack(_toeplitz_w(conv2_w, 12, 8, 32, 64), 8, 64, 256, 512)
    b2t = jnp.tile(conv2_b.reshape(1, 64), (1, 4))
    w3c = _toeplitz_w(conv3_w, 4, 3, 64, 128)[:, 0:256]
    b3 = conv3_b.reshape(1, 128)
    wf1 = fc1_w.reshape(128, 9, 512).sum(axis=1)
    bf1 = fc1_b.reshape(1, 512)
    wf2 = fc2_w
    bf2 = fc2_b.reshape(1, 10)

    def const2d(shape):
        return pl.BlockSpec(shape, lambda g: (0, 0))

    out = pl.pallas_call(
        _fused_kernel,
        out_shape=jax.ShapeDtypeStruct((n_pad, 10), jnp.float32),
        grid=(n_pad // Bb,),
        in_specs=[
            pl.BlockSpec((7, Bb, 128), lambda g: (0, g, 0)),
            pl.BlockSpec((7, Bb, 128), lambda g: (0, g, 0)),
            pl.BlockSpec((7, Bb, 128), lambda g: (0, g, 0)),
            pl.BlockSpec((7, Bb, 128), lambda g: (0, g, 0)),
            const2d((384, 768)),
            const2d((1, 384)),
            const2d((1152, 512)),
            const2d((1, 256)),
            const2d((768, 256)),
            const2d((1, 128)),
            const2d((128, 512)),
            const2d((1, 512)),
            const2d((512, 10)),
            const2d((1, 10)),
        ],
        out_specs=pl.BlockSpec((Bb, 10), lambda g: (g, 0)),
        compiler_params=pltpu.CompilerParams(
            dimension_semantics=("parallel",),
            vmem_limit_bytes=60 * 1024 * 1024,
        ),
    )(*xqs, w1c, b1t, w2c, b2t, w3c, b3, wf1, bf1, wf2, bf2)
    return out[:N]


# speedup vs baseline: 6.1970x; 1.0453x over previous
import jax
import jax.numpy as jnp
from jax.experimental import pallas as pl
from jax.experimental.pallas import tpu as pltpu

_BB = 256


def _fused_kernel(x0_ref, x1_ref, x2_ref, x3_ref, w1c_ref, b1t_ref,
                  w2c_ref, b2t_ref, w3c_ref, b3_ref, wf1_ref, bf1_ref,
                  wf2_ref, bf2_ref, o_ref):
    Bb = x0_ref.shape[1]
    xq = [x0_ref[...], x1_ref[...], x2_ref[...], x3_ref[...]]

    def dotf(a, w):
        return jnp.dot(a, w, preferred_element_type=jnp.float32)

    w1c = w1c_ref[...]
    b1t = b1t_ref[...]
    xa = jnp.concatenate([xq[0][0:5], xq[1][0:5], xq[2][0:5]], -1)
    xb = jnp.concatenate([xq[1][0:5], xq[2][0:5], xq[3][0:5]], -1)
    xc = jnp.concatenate([xq[2][0:5], xq[3][0:5], xq[0][1:6]], -1)
    xd = jnp.concatenate([xq[3][0:5], xq[0][1:6], xq[1][1:6]], -1)
    ya = dotf(xa.reshape(5 * Bb, 384), w1c)
    yb = dotf(xb.reshape(5 * Bb, 384), w1c)
    yc = dotf(xc.reshape(5 * Bb, 384), w1c)
    yd = dotf(xd.reshape(5 * Bb, 384), w1c)
    pe = jnp.maximum(jnp.maximum(ya[:, 0:384], ya[:, 384:768]),
                     jnp.maximum(yb[:, 0:384], yb[:, 384:768]))
    po = jnp.maximum(jnp.maximum(yc[:, 0:384], yc[:, 384:768]),
                     jnp.maximum(yd[:, 0:384], yd[:, 384:768]))
    p1e = jnp.maximum(pe + b1t, 0.0)
    p1o = jnp.maximum(po + b1t, 0.0)

    w2c = w2c_ref[...]
    b2t = b2t_ref[...]
    x2a = jnp.concatenate([p1e[0:4 * Bb], p1o[0:4 * Bb], p1e[Bb:5 * Bb]], -1)
    x2b = jnp.concatenate([p1o[0:4 * Bb], p1e[Bb:5 * Bb], p1o[Bb:5 * Bb]], -1)
    y2a = dotf(x2a, w2c)
    y2b = dotf(x2b, w2c)
    q2 = jnp.maximum(jnp.maximum(y2a[:, 0:256], y2a[:, 256:512]),
                     jnp.maximum(y2b[:, 0:256], y2b[:, 256:512]))
    p2 = jnp.maximum(q2 + b2t, 0.0)

    w3c = w3c_ref[...]
    x3 = jnp.concatenate([p2[0:2 * Bb], p2[Bb:3 * Bb], p2[2 * Bb:4 * Bb]], -1)
    y3 = dotf(x3, w3c)
    v = jnp.maximum(y3[0:Bb], y3[Bb:2 * Bb])
    f = jnp.maximum(jnp.maximum(v[:, 0:128], v[:, 128:256])
                    + b3_ref[...], 0.0)

    h = jnp.maximum(dotf(f, wf1_ref[...]) + bf1_ref[...], 0.0)
    logits = dotf(h, wf2_ref[...]) + bf2_ref[...]
    m = jnp.max(logits, axis=-1, keepdims=True)
    s = logits - m
    lse = jnp.log(jnp.sum(jnp.exp(s), axis=-1, keepdims=True))
    o_ref[...] = (s - lse).astype(o_ref.dtype)


def _toeplitz_w(w, win, wout, cin, cout):
    diff = jnp.arange(win)[:, None] - jnp.arange(wout)[None, :]
    mask = (diff >= 0) & (diff <= 2)
    idx = jnp.clip(diff, 0, 2)
    wt = w[:, idx]
    wt = jnp.where(mask[None, :, :, None, None], wt, 0.0)
    wt = wt.transpose(0, 1, 3, 2, 4)
    return wt.reshape(3 * win * cin, wout * cout)


def _parity_pack(wt, wout, cout, off_odd, n_total):
    k = wt.shape[0]
    w3 = wt.reshape(k, wout, cout)
    we = w3[:, 0::2].reshape(k, -1)
    wo = w3[:, 1::2].reshape(k, -1)
    out = jnp.zeros((k, n_total), wt.dtype)
    out = out.at[:, 0:we.shape[1]].set(we)
    out = out.at[:, off_odd:off_odd + wo.shape[1]].set(wo)
    return out


def kernel(x_nchw, conv1_w, conv1_b, conv2_w, conv2_b, conv3_w, conv3_b,
           fc1_w, fc1_b, fc2_w, fc2_b):
    N, C, H, W = x_nchw.shape
    assert (C, H, W) == (1, 28, 28), (C, H, W)
    Bb = _BB
    n_pad = int(pl.cdiv(N, Bb)) * Bb

    x = x_nchw.reshape(N, 28, 28)
    if n_pad != N:
        x = jnp.pad(x, ((0, n_pad - N), (0, 0), (0, 0)))
    xqs = [jnp.pad(x[:, r::4, :], ((0, 0), (0, 0), (0, 100))).transpose(1, 0, 2)
           for r in range(4)]

    w1t = _toeplitz_w(conv1_w.reshape(3, 3, 1, 32), 28, 26, 1, 32)
    w1t = w1t.reshape(84, 26, 32)[:, 0:20].reshape(84, 640)
    w1p = _parity_pack(w1t, 20, 32, 384, 768)
    w1c = jnp.zeros((3, 128, 768), w1p.dtype).at[:, 0:28].set(
        w1p.reshape(3, 28, 768)).reshape(384, 768)
    b1t = jnp.zeros((1, 384), jnp.float32).at[:, 0:320].set(
        jnp.tile(conv1_b.reshape(1, 32), (1, 10)))
    w2c = _parity_pack(_toeplitz_w(conv2_w, 12, 8, 32, 64), 8, 64, 256, 512)
    b2t = jnp.tile(conv2_b.reshape(1, 64), (1, 4))
    w3c = _toeplitz_w(conv3_w, 4, 3, 64, 128)[:, 0:256]
    b3 = conv3_b.reshape(1, 128)
    wf1 = fc1_w.reshape(128, 9, 512).sum(axis=1)
    bf1 = fc1_b.reshape(1, 512)
    wf2 = fc2_w
    bf2 = fc2_b.reshape(1, 10)

    def const2d(shape):
        return pl.BlockSpec(shape, lambda g: (0, 0))

    out = pl.pallas_call(
        _fused_kernel,
        out_shape=jax.ShapeDtypeStruct((n_pad, 10), jnp.float32),
        grid=(n_pad // Bb,),
        in_specs=[
            pl.BlockSpec((7, Bb, 128), lambda g: (0, g, 0)),
            pl.BlockSpec((7, Bb, 128), lambda g: (0, g, 0)),
            pl.BlockSpec((7, Bb, 128), lambda g: (0, g, 0)),
            pl.BlockSpec((7, Bb, 128), lambda g: (0, g, 0)),
            const2d((384, 768)),
            const2d((1, 384)),
            const2d((1152, 512)),
            const2d((1, 256)),
            const2d((768, 256)),
            const2d((1, 128)),
            const2d((128, 512)),
            const2d((1, 512)),
            const2d((512, 10)),
            const2d((1, 10)),
        ],
        out_specs=pl.BlockSpec((Bb, 10), lambda g: (g, 0)),
        compiler_params=pltpu.CompilerParams(
            dimension_semantics=("parallel",),
            vmem_limit_bytes=60 * 1024 * 1024,
        ),
    )(*xqs, w1c, b1t, w2c, b2t, w3c, b3, wf1, bf1, wf2, bf2)
    return out[:N]
